# baseline (speedup 1.0000x reference)
import jax
import jax.numpy as jnp
from jax.experimental import pallas as pl
from jax.experimental.pallas import tpu as pltpu

N = 8192
D = 512
BC = 512
NB = N // BC
RC = 2048
_SQRT2 = 1.4142135623730951


def _emb_loss_kernel(w_ref, out_ref, rowacc, colacc):
    w_full = w_ref[...]
    w2 = w_full * w_full

    sq_row = jax.lax.dot_general(
        jnp.ones((1, D), jnp.float32), w2,
        dimension_numbers=(((1,), (1,)), ((), ())),
        preferred_element_type=jnp.float32,
    )
    sq_col = jax.lax.dot_general(
        w2, jnp.ones((D, 1), jnp.float32),
        dimension_numbers=(((1,), (0,)), ((), ())),
        preferred_element_type=jnp.float32,
    )

    wsbf = (w_full * _SQRT2).astype(jnp.bfloat16)

    sq_row_bf = sq_row.astype(jnp.bfloat16)
    sq_col_bf = sq_col.astype(jnp.bfloat16)

    rowacc[...] = jnp.full((N, 128), jnp.inf, jnp.bfloat16)

    r = jax.lax.broadcasted_iota(jnp.int32, (BC, BC), 0)
    c = jax.lax.broadcasted_iota(jnp.int32, (BC, BC), 1)
    eye_inf = jnp.where(r == c, jnp.inf, 0.0).astype(jnp.bfloat16)

    for j in range(NB):
        cl, chi = j * BC, (j + 1) * BC
        rhs = wsbf[cl:chi, :]
        sq_c_blk = sq_row_bf[:, cl:chi]
        colmin = None

        pieces = [(base, min(RC, cl - base), False)
                  for base in range(0, cl, RC)]
        pieces.append((cl, BC, True))

        for base, sz, is_diag in pieces:
            g = jax.lax.dot_general(
                wsbf[base:base + sz, :], rhs,
                dimension_numbers=(((1,), (1,)), ((), ())),
                preferred_element_type=jnp.float32,
            ).astype(jnp.bfloat16)
            if is_diag:
                g = g - eye_inf
            t1 = sq_col_bf[base:base + sz, :] - g
            part_col = jnp.min(t1, axis=0, keepdims=True)
            colmin = (part_col if colmin is None
                      else jnp.minimum(colmin, part_col))
            t2 = sq_c_blk - g
            part_row = jnp.minimum(
                jnp.minimum(t2[:, 0:128], t2[:, 128:256]),
                jnp.minimum(t2[:, 256:384], t2[:, 384:512]))
            rowacc[base:base + sz, :] = jnp.minimum(
                rowacc[base:base + sz, :], part_row)
        colacc[:, cl:chi] = colmin

    rowmin = jnp.min(rowacc[...], axis=1, keepdims=True)
    m = jnp.minimum(rowmin, jnp.reshape(colacc[...], (N, 1)))
    min_d2 = sq_col + m.astype(jnp.float32)
    d = jnp.sqrt(jnp.maximum(min_d2, 1e-12))
    mean = jnp.sum(d) / N
    kept = jnp.where(d > mean, 0.0, d)
    loss = -(jnp.sum(kept) / N)
    out_ref[0] = loss
    out_ref[1] = mean


def kernel(weight):
    out = pl.pallas_call(
        _emb_loss_kernel,
        in_specs=[pl.BlockSpec((N, D), lambda: (0, 0))],
        out_specs=pl.BlockSpec(memory_space=pltpu.SMEM),
        out_shape=jax.ShapeDtypeStruct((2,), jnp.float32),
        scratch_shapes=[
            pltpu.VMEM((N, 128), jnp.bfloat16),
            pltpu.VMEM((1, N), jnp.bfloat16),
        ],
    )(weight)
    return out


# speedup vs baseline: 1.0054x; 1.0054x over previous
import jax
import jax.numpy as jnp
from jax.experimental import pallas as pl
from jax.experimental.pallas import tpu as pltpu

N = 8192
D = 512
BC = 512
NB = N // BC
RC = 2048
_SQRT2 = 1.4142135623730951


def _emb_loss_kernel(w_ref, out_ref, rowacc, colacc):
    w_full = w_ref[...]
    w2 = w_full * w_full

    sq_row = jax.lax.dot_general(
        jnp.ones((1, D), jnp.float32), w2,
        dimension_numbers=(((1,), (1,)), ((), ())),
        preferred_element_type=jnp.float32,
    )
    sq_col = jax.lax.dot_general(
        w2, jnp.ones((D, 1), jnp.float32),
        dimension_numbers=(((1,), (0,)), ((), ())),
        preferred_element_type=jnp.float32,
    )

    wsbf = (w_full * _SQRT2).astype(jnp.bfloat16)

    sq_row_bf = sq_row.astype(jnp.bfloat16)
    sq_col_bf = sq_col.astype(jnp.bfloat16)

    rowacc[...] = jnp.full((N, 128), jnp.inf, jnp.float32)

    r = jax.lax.broadcasted_iota(jnp.int32, (BC, BC), 0)
    c = jax.lax.broadcasted_iota(jnp.int32, (BC, BC), 1)
    eye_inf = jnp.where(r == c, jnp.inf, 0.0).astype(jnp.bfloat16)

    for j in range(NB):
        cl, chi = j * BC, (j + 1) * BC
        rhs = wsbf[cl:chi, :]
        sq_c_blk = sq_row_bf[:, cl:chi]
        colmin = None

        pieces = [(base, min(RC, cl - base), False)
                  for base in range(0, cl, RC)]
        pieces.append((cl, BC, True))

        for base, sz, is_diag in pieces:
            g = jax.lax.dot_general(
                wsbf[base:base + sz, :], rhs,
                dimension_numbers=(((1,), (1,)), ((), ())),
                preferred_element_type=jnp.float32,
            ).astype(jnp.bfloat16)
            if is_diag:
                g = g - eye_inf
            t = (sq_col_bf[base:base + sz, :] - g) + sq_c_blk
            part_col = jnp.min(t, axis=0, keepdims=True)
            colmin = (part_col if colmin is None
                      else jnp.minimum(colmin, part_col))
            part_row = jnp.minimum(
                jnp.minimum(t[:, 0:128], t[:, 128:256]),
                jnp.minimum(t[:, 256:384], t[:, 384:512]))
            rowacc[base:base + sz, :] = jnp.minimum(
                rowacc[base:base + sz, :], part_row.astype(jnp.float32))
        colacc[:, cl:chi] = colmin.astype(jnp.float32)

    rowmin = jnp.min(rowacc[...], axis=1, keepdims=True)
    min_d2 = jnp.minimum(rowmin, jnp.reshape(colacc[...], (N, 1)))
    d = jnp.sqrt(jnp.maximum(min_d2, 1e-12))
    mean = jnp.sum(d) / N
    kept = jnp.where(d > mean, 0.0, d)
    loss = -(jnp.sum(kept) / N)
    out_ref[0] = loss
    out_ref[1] = mean


def kernel(weight):
    out = pl.pallas_call(
        _emb_loss_kernel,
        in_specs=[pl.BlockSpec((N, D), lambda: (0, 0))],
        out_specs=pl.BlockSpec(memory_space=pltpu.SMEM),
        out_shape=jax.ShapeDtypeStruct((2,), jnp.float32),
        scratch_shapes=[
            pltpu.VMEM((N, 128), jnp.float32),
            pltpu.VMEM((1, N), jnp.float32),
        ],
    )(weight)
    return out


# speedup vs baseline: 1.0131x; 1.0077x over previous
import jax
import jax.numpy as jnp
from jax.experimental import pallas as pl
from jax.experimental.pallas import tpu as pltpu

N = 8192
D = 512
BC = 512
NB = N // BC
RC = 2048
_SQRT2 = 1.4142135623730951


def _emb_loss_kernel(w_ref, out_ref, rowacc, colacc):
    w_full = w_ref[...]
    w2 = w_full * w_full

    sq_row = jax.lax.dot_general(
        jnp.ones((1, D), jnp.float32), w2,
        dimension_numbers=(((1,), (1,)), ((), ())),
        preferred_element_type=jnp.float32,
    )
    sq_col = jax.lax.dot_general(
        w2, jnp.ones((D, 1), jnp.float32),
        dimension_numbers=(((1,), (0,)), ((), ())),
        preferred_element_type=jnp.float32,
    )

    wsbf = (w_full * _SQRT2).astype(jnp.bfloat16)

    sq_row_bf = sq_row.astype(jnp.bfloat16)
    sq_col_bf = sq_col.astype(jnp.bfloat16)

    rowacc[...] = jnp.full((N, 128), jnp.inf, jnp.float32)

    r = jax.lax.broadcasted_iota(jnp.int32, (BC, BC), 0)
    c = jax.lax.broadcasted_iota(jnp.int32, (BC, BC), 1)
    eye_inf = jnp.where(r == c, jnp.inf, 0.0).astype(jnp.bfloat16)

    for j in range(NB):
        cl, chi = j * BC, (j + 1) * BC
        rhs = wsbf[cl:chi, :]
        sq_c_blk = sq_row_bf[:, cl:chi]
        colmin = None

        pieces = [(base, min(RC, cl - base), False)
                  for base in range(0, cl, RC)]
        pieces.append((cl, BC, True))

        for base, sz, is_diag in pieces:
            g = jax.lax.dot_general(
                wsbf[base:base + sz, :], rhs,
                dimension_numbers=(((1,), (1,)), ((), ())),
                preferred_element_type=jnp.float32,
            ).astype(jnp.bfloat16)
            if is_diag:
                g = g - eye_inf
            t1 = sq_col_bf[base:base + sz, :] - g
            part_col = jnp.min(t1, axis=0, keepdims=True)
            colmin = (part_col if colmin is None
                      else jnp.minimum(colmin, part_col))
            t2 = sq_c_blk - g
            part_row = jnp.minimum(
                jnp.minimum(t2[:, 0:128], t2[:, 128:256]),
                jnp.minimum(t2[:, 256:384], t2[:, 384:512]))
            rowacc[base:base + sz, :] = jnp.minimum(
                rowacc[base:base + sz, :], part_row.astype(jnp.float32))
        colacc[:, cl:chi] = colmin.astype(jnp.float32)

    rowmin = jnp.min(rowacc[...], axis=1, keepdims=True)
    m = jnp.minimum(rowmin, jnp.reshape(colacc[...], (N, 1)))
    min_d2 = sq_col + m
    d = jnp.sqrt(jnp.maximum(min_d2, 1e-12))
    mean = jnp.sum(d) / N
    kept = jnp.where(d > mean, 0.0, d)
    loss = -(jnp.sum(kept) / N)
    out_ref[0] = loss
    out_ref[1] = mean


def kernel(weight):
    out = pl.pallas_call(
        _emb_loss_kernel,
        in_specs=[pl.BlockSpec((N, D), lambda: (0, 0))],
        out_specs=pl.BlockSpec(memory_space=pltpu.SMEM),
        out_shape=jax.ShapeDtypeStruct((2,), jnp.float32),
        scratch_shapes=[
            pltpu.VMEM((N, 128), jnp.float32),
            pltpu.VMEM((1, N), jnp.float32),
        ],
    )(weight)
    return out
